# baseline (speedup 1.0000x reference)
import jax
import jax.numpy as jnp
from jax.experimental import pallas as pl
from jax.experimental.pallas import tpu as pltpu

_VB = 512
_WB = 512


def _body(x_ref, a_ref, p_ref, out_ref, lhs_ref, xtf_ref, acc_ref):
    w = pl.program_id(0)
    v = pl.program_id(1)
    nv = pl.num_programs(1)
    nbc = x_ref.shape[0]
    ll = x_ref.shape[2]
    half = nbc // 2 * ll

    @pl.when(w == 0)
    def _build_chunk():
        xc = x_ref[...]
        pieces = []
        for c in range(2):
            for b in range(nbc // 2):
                pieces.append(xc[b * 2 + c].T)
        chunk = jnp.concatenate(pieces, axis=0)
        xtf_ref[v] = chunk
        lhs_ref[v] = jnp.where(jnp.isneginf(chunk), 0.0, chunk).astype(jnp.bfloat16)

    @pl.when(v == 0)
    def _zero():
        acc_ref[...] = jnp.zeros_like(acc_ref)

    acc_ref[...] += jnp.dot(
        lhs_ref[v],
        a_ref[...].astype(jnp.bfloat16),
        preferred_element_type=jnp.float32,
    )

    @pl.when(v == nv - 1)
    def _epilogue():
        xt = xtf_ref[w]
        mask = jnp.isneginf(xt)
        imp = jnp.where(mask, 0.0, xt)
        acc = acc_ref[...]
        imp0, imp1 = imp[:half], imp[half:]
        x10, x11 = acc[:half], acc[half:]
        g0 = (p_ref[0, 0] * imp0 + p_ref[0, 1] * imp1
              + p_ref[0, 2] * x10 + p_ref[0, 3] * x11 + p_ref[0, 4])
        g1 = (p_ref[1, 0] * imp0 + p_ref[1, 1] * imp1
              + p_ref[1, 2] * x10 + p_ref[1, 3] * x11 + p_ref[1, 4])
        gcn = jnp.concatenate([g0, g1], axis=0)
        ot = jnp.where(mask, gcn, imp)
        pieces = []
        for b in range(nbc // 2):
            for c in range(2):
                r0 = c * half + b * ll
                pieces.append(ot[r0:r0 + ll, :].T)
        out_ref[...] = jnp.stack(pieces, axis=0)


def kernel(x, supports, W, b):
    B, C, N, L = x.shape
    R = C * B * L
    a = supports[0]
    xr = x.reshape(B * C, N, L)
    params = jnp.concatenate([W, b[:, None]], axis=1)
    nv = N // _VB
    nw = N // _WB

    out_n = pl.pallas_call(
        _body,
        grid=(nw, nv),
        in_specs=[
            pl.BlockSpec(
                (B * C, _VB, L),
                lambda w, v, _nv=nv: (0, jnp.where(w == 0, v, _nv - 1), 0),
            ),
            pl.BlockSpec((_VB, _WB), lambda w, v: (v, w)),
            pl.BlockSpec(memory_space=pltpu.SMEM),
        ],
        out_specs=pl.BlockSpec((B * C, _WB, L), lambda w, v: (0, w, 0)),
        out_shape=jax.ShapeDtypeStruct((B * C, N, L), jnp.float32),
        scratch_shapes=[
            pltpu.VMEM((nv, R, _VB), jnp.bfloat16),
            pltpu.VMEM((nv, R, _VB), jnp.float32),
            pltpu.VMEM((R, _WB), jnp.float32),
        ],
    )(xr, a, params)

    return out_n.reshape(B, C, N, L)


# speedup vs baseline: 1.2737x; 1.2737x over previous
import functools

import jax
import jax.numpy as jnp
from jax.experimental import pallas as pl
from jax.experimental.pallas import tpu as pltpu

_VB = 512


def _body(x_ref, a_ref, p_ref, out_ref, xtf_ref, *, bdim, cdim, ldim):
    v = pl.program_id(0)
    nv = pl.num_programs(0)
    half = xtf_ref.shape[1] // 2

    xc = x_ref[...]
    chunk = jnp.transpose(
        xc.reshape(bdim, cdim, _VB, ldim), (1, 0, 3, 2)
    ).reshape(cdim * bdim * ldim, _VB)
    xtf_ref[v] = chunk
    impc = jnp.where(jnp.isneginf(chunk), 0.0, chunk).astype(jnp.bfloat16)
    contrib = jnp.dot(
        impc,
        a_ref[...].astype(jnp.bfloat16),
        preferred_element_type=jnp.float32,
    )

    @pl.when(v == 0)
    def _init():
        out_ref[...] = contrib

    @pl.when(v != 0)
    def _acc():
        out_ref[...] += contrib

    @pl.when(v == nv - 1)
    def _epilogue():
        xt = jnp.concatenate([xtf_ref[i] for i in range(nv)], axis=1)
        mask = jnp.isneginf(xt)
        imp = jnp.where(mask, 0.0, xt)
        acc = out_ref[...]
        imp0, imp1 = imp[:half], imp[half:]
        x10, x11 = acc[:half], acc[half:]
        g0 = (p_ref[0, 0] * imp0 + p_ref[0, 1] * imp1
              + p_ref[0, 2] * x10 + p_ref[0, 3] * x11 + p_ref[0, 4])
        g1 = (p_ref[1, 0] * imp0 + p_ref[1, 1] * imp1
              + p_ref[1, 2] * x10 + p_ref[1, 3] * x11 + p_ref[1, 4])
        gcn = jnp.concatenate([g0, g1], axis=0)
        out_ref[...] = jnp.where(mask, gcn, imp)


def kernel(x, supports, W, b):
    B, C, N, L = x.shape
    R = C * B * L
    nv = N // _VB
    a = supports[0]
    x2 = x.reshape(B * C, N * L)
    params = jnp.concatenate([W, b[:, None]], axis=1)

    out_t = pl.pallas_call(
        functools.partial(_body, bdim=B, cdim=C, ldim=L),
        grid=(nv,),
        in_specs=[
            pl.BlockSpec((B * C, _VB * L), lambda v: (0, v)),
            pl.BlockSpec((_VB, N), lambda v: (v, 0)),
            pl.BlockSpec(memory_space=pltpu.SMEM),
        ],
        out_specs=pl.BlockSpec((R, N), lambda v: (0, 0)),
        out_shape=jax.ShapeDtypeStruct((R, N), jnp.float32),
        scratch_shapes=[pltpu.VMEM((nv, R, _VB), jnp.float32)],
    )(x2, a, params)

    return out_t.reshape(C, B, L, N).transpose(1, 0, 3, 2)


# speedup vs baseline: 2.1916x; 1.7207x over previous
import jax
import jax.numpy as jnp
from jax.experimental import pallas as pl
from jax.experimental.pallas import tpu as pltpu

_KB = 1024
_NW = 2


def _body(xc_ref, a_ref, xw_ref, p_ref, out_ref):
    v = pl.program_id(1)
    nv = pl.num_programs(1)

    xc = xc_ref[...]
    impc = jnp.where(jnp.isneginf(xc), 0.0, xc).astype(jnp.bfloat16)
    contrib = jnp.dot(
        impc,
        a_ref[...].astype(jnp.bfloat16),
        preferred_element_type=jnp.float32,
    )

    @pl.when(v == 0)
    def _init():
        out_ref[...] = contrib

    @pl.when(v != 0)
    def _acc():
        out_ref[...] += contrib

    @pl.when(v == nv - 1)
    def _epilogue():
        xt = xw_ref[...]
        mask = jnp.isneginf(xt)
        imp = jnp.where(mask, 0.0, xt)
        acc = out_ref[...]
        half = imp.shape[0] // 2
        imp0, imp1 = imp[:half], imp[half:]
        x10, x11 = acc[:half], acc[half:]
        g0 = (p_ref[0, 0] * imp0 + p_ref[0, 1] * imp1
              + p_ref[0, 2] * x10 + p_ref[0, 3] * x11 + p_ref[0, 4])
        g1 = (p_ref[1, 0] * imp0 + p_ref[1, 1] * imp1
              + p_ref[1, 2] * x10 + p_ref[1, 3] * x11 + p_ref[1, 4])
        gcn = jnp.concatenate([g0, g1], axis=0)
        out_ref[...] = jnp.where(mask, gcn, imp)


def kernel(x, supports, W, b):
    B, C, N, L = x.shape
    R = C * B * L
    a = supports[0]
    wb = N // _NW
    xt = jnp.transpose(x, (1, 0, 3, 2)).reshape(R, N)
    params = jnp.concatenate([W, b[:, None]], axis=1)

    out_t = pl.pallas_call(
        _body,
        grid=(_NW, N // _KB),
        in_specs=[
            pl.BlockSpec((R, _KB), lambda w, v: (0, v)),
            pl.BlockSpec((_KB, wb), lambda w, v: (v, w)),
            pl.BlockSpec((R, wb), lambda w, v: (0, w)),
            pl.BlockSpec(memory_space=pltpu.SMEM),
        ],
        out_specs=pl.BlockSpec((R, wb), lambda w, v: (0, w)),
        out_shape=jax.ShapeDtypeStruct((R, N), jnp.float32),
    )(xt, a, xt, params)

    return out_t.reshape(C, B, L, N).transpose(1, 0, 3, 2)
